# baseline (speedup 1.0000x reference)
import functools

import jax
import jax.numpy as jnp
from jax.experimental import pallas as pl
from jax.experimental.pallas import tpu as pltpu

_BM = 1024
_BN = 8192
_D = 128

_NEG = float("-inf")


def _topk_kernel(q_ref, k_ref, o_ref, kbs, qbs, r1, r2, r3, r4, *, tail_len, bn, nkb):
    j = pl.program_id(0)
    i = pl.program_id(1)

    @pl.when(j == 0)
    def _init():
        neg = jnp.full((r1.shape[1], r1.shape[2]), _NEG, jnp.float32)
        r1[i] = neg
        r2[i] = neg
        r3[i] = neg
        r4[i] = neg

    @pl.when(i == 0)
    def _norm_keys():
        k = k_ref[...]
        kn = jnp.sqrt(jnp.sum(k * k, axis=1, keepdims=True))
        kbs[...] = k / kn

    @pl.when(j == 0)
    def _norm_queries():
        q = q_ref[...]
        qn = jnp.sqrt(jnp.sum(q * q, axis=1, keepdims=True))
        qbs[i] = q / qn

    s = jax.lax.dot_general(
        qbs[i], kbs[...], (((1,), (1,)), ((), ())),
        preferred_element_type=jnp.float32,
    )

    mask_start = tail_len // _D
    full_start = -(-tail_len // _D)
    lane = jax.lax.broadcasted_iota(jnp.int32, (8, _D), 1)[0:1]

    def _maybe_mask(v, ch):
        if ch >= mask_start:
            col = lane + (j * bn + ch * _D)
            if ch >= full_start:
                keep = col >= bn
            else:
                keep = (col < tail_len) | (col >= bn)
            v = jnp.where(keep, v, _NEG)
        return v

    a, b, c = r1[i], r2[i], r3[i]
    d = r4[i]
    for p in range(bn // (2 * _D)):
        v1 = _maybe_mask(s[:, (2 * p) * _D:(2 * p + 1) * _D], 2 * p)
        v2 = _maybe_mask(s[:, (2 * p + 1) * _D:(2 * p + 2) * _D], 2 * p + 1)
        hi = jnp.maximum(v1, v2)
        lo = jnp.minimum(v1, v2)
        m1 = jnp.minimum(a, hi)
        a = jnp.maximum(a, hi)
        m2 = jnp.minimum(b, m1)
        b = jnp.maximum(b, m1)
        c = jnp.maximum(c, m2)
        d = jnp.maximum(d, lo)
    r1[i], r2[i], r3[i], r4[i] = a, b, c, d

    @pl.when(j == nkb - 1)
    def _done():
        x = jnp.concatenate([r1[i], r2[i], r3[i], r4[i]], axis=1)
        b1 = jnp.max(x, axis=1, keepdims=True)
        eq1 = x == b1
        c1 = jnp.sum(eq1.astype(jnp.float32), axis=1, keepdims=True)
        x2 = jnp.where(eq1, _NEG, x)
        b2 = jnp.max(x2, axis=1, keepdims=True)
        eq2 = x2 == b2
        c2 = jnp.sum(eq2.astype(jnp.float32), axis=1, keepdims=True)
        x3 = jnp.where(eq2, _NEG, x2)
        b3 = jnp.max(x3, axis=1, keepdims=True)

        t2 = jnp.where(c1 >= 2.0, b1, b2)
        t3 = jnp.where(
            c1 >= 3.0, b1, jnp.where(c1 == 2.0, b2, jnp.where(c2 >= 2.0, b2, b3))
        )
        mean = (b1 + t2 + t3) * jnp.float32(1.0 / 3.0)
        o_ref[...] = jnp.broadcast_to(mean, o_ref.shape)


def kernel(tensor_1, tensor_2):
    m, d = tensor_1.shape
    n_keys = tensor_2.shape[0]

    nkb = (n_keys + _BN - 1) // _BN
    tail_len = n_keys - (nkb - 1) * _BN
    nqb = m // _BM

    out = pl.pallas_call(
        functools.partial(_topk_kernel, tail_len=tail_len, bn=_BN, nkb=nkb),
        grid=(nkb, nqb),
        in_specs=[
            pl.BlockSpec((_BM, d), lambda j, i: (i, 0)),
            pl.BlockSpec((_BN, d), lambda j, i: ((j + nkb - 1) % nkb, 0)),
        ],
        out_specs=pl.BlockSpec((_BM, _D), lambda j, i: (i, 0)),
        out_shape=jax.ShapeDtypeStruct((m, _D), jnp.float32),
        scratch_shapes=[
            pltpu.VMEM((_BN, _D), jnp.float32),
            pltpu.VMEM((nqb, _BM, _D), jnp.float32),
            pltpu.VMEM((nqb, _BM, _D), jnp.float32),
            pltpu.VMEM((nqb, _BM, _D), jnp.float32),
            pltpu.VMEM((nqb, _BM, _D), jnp.float32),
            pltpu.VMEM((nqb, _BM, _D), jnp.float32),
        ],
        compiler_params=pltpu.CompilerParams(
            dimension_semantics=("arbitrary", "arbitrary"),
        ),
    )(tensor_1, tensor_2)
    return out[:, 0]


# speedup vs baseline: 1.1193x; 1.1193x over previous
import functools

import jax
import jax.numpy as jnp
from jax.experimental import pallas as pl
from jax.experimental.pallas import tpu as pltpu

_BM = 2048
_BN = 4096
_D = 128

_NEG = float("-inf")


def _topk_kernel(q_ref, k_ref, o_ref, kbs, qbs, r1, r2, r3, r4, *, tail_len, bn, nkb):
    j = pl.program_id(0)
    i = pl.program_id(1)

    @pl.when(j == 0)
    def _init():
        neg = jnp.full((r1.shape[1], r1.shape[2]), _NEG, jnp.float32)
        r1[i] = neg
        r2[i] = neg
        r3[i] = neg
        r4[i] = neg

    @pl.when(i == 0)
    def _norm_keys():
        k = k_ref[...]
        kn = jnp.sqrt(jnp.sum(k * k, axis=1, keepdims=True))
        kbs[...] = k / kn

    @pl.when(j == 0)
    def _norm_queries():
        q = q_ref[...]
        qn = jnp.sqrt(jnp.sum(q * q, axis=1, keepdims=True))
        qbs[i] = q / qn

    s = jax.lax.dot_general(
        qbs[i], kbs[...], (((1,), (1,)), ((), ())),
        preferred_element_type=jnp.float32,
    )

    mask_start = tail_len // _D
    full_start = -(-tail_len // _D)
    lane = jax.lax.broadcasted_iota(jnp.int32, (8, _D), 1)[0:1]

    def _maybe_mask(v, ch):
        if ch >= mask_start:
            col = lane + (j * bn + ch * _D)
            if ch >= full_start:
                keep = col >= bn
            else:
                keep = (col < tail_len) | (col >= bn)
            v = jnp.where(keep, v, _NEG)
        return v

    a, b, c = r1[i], r2[i], r3[i]
    d = r4[i]
    for p in range(bn // (2 * _D)):
        v1 = _maybe_mask(s[:, (2 * p) * _D:(2 * p + 1) * _D], 2 * p)
        v2 = _maybe_mask(s[:, (2 * p + 1) * _D:(2 * p + 2) * _D], 2 * p + 1)
        hi = jnp.maximum(v1, v2)
        lo = jnp.minimum(v1, v2)
        m1 = jnp.minimum(a, hi)
        a = jnp.maximum(a, hi)
        m2 = jnp.minimum(b, m1)
        b = jnp.maximum(b, m1)
        c = jnp.maximum(c, m2)
        d = jnp.maximum(d, lo)
    r1[i], r2[i], r3[i], r4[i] = a, b, c, d

    @pl.when(j == nkb - 1)
    def _done():
        x = jnp.concatenate([r1[i], r2[i], r3[i], r4[i]], axis=1)
        b1 = jnp.max(x, axis=1, keepdims=True)
        eq1 = x == b1
        c1 = jnp.sum(eq1.astype(jnp.float32), axis=1, keepdims=True)
        x2 = jnp.where(eq1, _NEG, x)
        b2 = jnp.max(x2, axis=1, keepdims=True)
        eq2 = x2 == b2
        c2 = jnp.sum(eq2.astype(jnp.float32), axis=1, keepdims=True)
        x3 = jnp.where(eq2, _NEG, x2)
        b3 = jnp.max(x3, axis=1, keepdims=True)

        t2 = jnp.where(c1 >= 2.0, b1, b2)
        t3 = jnp.where(
            c1 >= 3.0, b1, jnp.where(c1 == 2.0, b2, jnp.where(c2 >= 2.0, b2, b3))
        )
        mean = (b1 + t2 + t3) * jnp.float32(1.0 / 3.0)
        o_ref[...] = jnp.broadcast_to(mean, o_ref.shape)


def kernel(tensor_1, tensor_2):
    m, d = tensor_1.shape
    n_keys = tensor_2.shape[0]

    nkb = (n_keys + _BN - 1) // _BN
    tail_len = n_keys - (nkb - 1) * _BN
    nqb = m // _BM

    out = pl.pallas_call(
        functools.partial(_topk_kernel, tail_len=tail_len, bn=_BN, nkb=nkb),
        grid=(nkb, nqb),
        in_specs=[
            pl.BlockSpec((_BM, d), lambda j, i: (i, 0)),
            pl.BlockSpec((_BN, d), lambda j, i: ((j + nkb - 1) % nkb, 0)),
        ],
        out_specs=pl.BlockSpec((_BM, _D), lambda j, i: (i, 0)),
        out_shape=jax.ShapeDtypeStruct((m, _D), jnp.float32),
        scratch_shapes=[
            pltpu.VMEM((_BN, _D), jnp.float32),
            pltpu.VMEM((nqb, _BM, _D), jnp.float32),
            pltpu.VMEM((nqb, _BM, _D), jnp.float32),
            pltpu.VMEM((nqb, _BM, _D), jnp.float32),
            pltpu.VMEM((nqb, _BM, _D), jnp.float32),
            pltpu.VMEM((nqb, _BM, _D), jnp.float32),
        ],
        compiler_params=pltpu.CompilerParams(
            dimension_semantics=("arbitrary", "arbitrary"),
        ),
    )(tensor_1, tensor_2)
    return out[:, 0]
